# baseline (speedup 1.0000x reference)
import functools

import jax
import jax.numpy as jnp
from jax import lax
from jax.experimental import pallas as pl
from jax.experimental.pallas import tpu as pltpu
from jax.experimental.pallas import tpu_sc as plsc

_PAD_ROWS = 8


def _tc_body(nblk, total_elems, first, *refs):
    if first:
        (x_ref, e_ref, m_ref, xsq_ref, ind_ref, indg_ref, table_ref,
         counts_out, acc_out, counts_ref, esqc_ref, ebft_ref, ones_ref,
         acc_ref) = refs
    else:
        (x_ref, e_ref, m_ref, xsq_ref, counts_in, acc_in, ind_ref,
         indg_ref, diff_ref, eff_ref, counts_ref, esqc_ref, ebft_ref,
         ones_ref, acc_ref) = refs
    i = pl.program_id(0)
    ne = e_ref.shape[1]
    r_ = x_ref.shape[0]

    @pl.when(i == 0)
    def _init():
        if first:
            counts_ref[...] = jnp.zeros_like(counts_ref)
            acc_ref[0] = 0.0
            acc_ref[1] = 0.0
        else:
            counts_ref[...] = counts_in[...]
            acc_ref[0] = acc_in[0]
            acc_ref[1] = acc_in[1]
        e0 = e_ref[...]
        et = e0.T
        if first:
            table_ref[0:ne, :] = et
            table_ref[ne:, :] = jnp.zeros_like(table_ref[ne:, :])
        ebft_ref[...] = (et + et).astype(jnp.bfloat16)
        esq = jnp.sum(e0 * e0, axis=0, keepdims=True)
        esqc_ref[...] = esq.reshape(ne, 1)
        ones_ref[...] = jnp.ones_like(ones_ref)

    x = x_ref[...]
    st2 = jnp.dot(ebft_ref[...], x.astype(jnp.bfloat16).T,
                  preferred_element_type=jnp.float32)
    xsq = xsq_ref[0]
    s = (st2 - xsq) - esqc_ref[...]
    smax = jnp.max(s, axis=0, keepdims=True)
    eq = s == smax
    iota = lax.broadcasted_iota(jnp.int32, (ne, r_), 0)
    rev = jnp.where(eq, jnp.int32(ne) - iota, 0)
    ind = jnp.int32(ne) - jnp.max(rev, axis=0)
    ind_ref[0, 0, :] = ind

    mrow = m_ref[0]
    indg_ref[0, 0, :] = jnp.where(mrow[0] > 0.0, ind, jnp.int32(ne))

    w = jnp.where(eq, mrow, 0.0)
    counts_ref[...] += jnp.dot(w, ones_ref[...],
                               preferred_element_type=jnp.float32)
    acc_ref[0] += jnp.sum(mrow[0] * (-smax[0]))
    acc_ref[1] += jnp.sum(mrow[0])

    @pl.when(i == nblk - 1)
    def _fin():
        if first:
            counts_out[...] = counts_ref[...]
            acc_out[0] = acc_ref[0]
            acc_out[1] = acc_ref[1]
        else:
            diff_ref[...] = jnp.full((1, 1), acc_ref[0] / total_elems,
                                     jnp.float32)
            mcount = jnp.maximum(acc_ref[1], 1.0)
            mu = counts_ref[...] / mcount
            eff_ref[...] = (jnp.full((1, 1), 1.0, jnp.float32)
                            / jnp.sum(mu * mu))


def _tc_stage(flatten, embed, mask3, xsq3, block_rows, first,
              counts_in=None, acc_in=None):
    n, d = flatten.shape
    ne = embed.shape[1]
    nblk = n // block_rows
    total = float(2 * n * d)
    in_specs = [
        pl.BlockSpec((block_rows, d), lambda i: (i, 0)),
        pl.BlockSpec((d, ne), lambda i: (0, 0)),
        pl.BlockSpec((1, 1, block_rows), lambda i: (i, 0, 0)),
        pl.BlockSpec((1, 1, block_rows), lambda i: (i, 0, 0)),
    ]
    operands = [flatten, embed, mask3, xsq3]
    out_specs = [
        pl.BlockSpec((1, 1, block_rows), lambda i: (i, 0, 0)),
        pl.BlockSpec((1, 1, block_rows), lambda i: (i, 0, 0)),
    ]
    out_shape = [
        jax.ShapeDtypeStruct((nblk, 1, block_rows), jnp.int32),
        jax.ShapeDtypeStruct((nblk, 1, block_rows), jnp.int32),
    ]
    if first:
        out_specs += [
            pl.BlockSpec((ne + _PAD_ROWS, d), lambda i: (0, 0)),
            pl.BlockSpec((ne, 1), lambda i: (0, 0)),
            pl.BlockSpec(memory_space=pltpu.SMEM),
        ]
        out_shape += [
            jax.ShapeDtypeStruct((ne + _PAD_ROWS, d), jnp.float32),
            jax.ShapeDtypeStruct((ne, 1), jnp.float32),
            jax.ShapeDtypeStruct((2,), jnp.float32),
        ]
    else:
        in_specs += [
            pl.BlockSpec((ne, 1), lambda i: (0, 0)),
            pl.BlockSpec(memory_space=pltpu.SMEM),
        ]
        operands += [counts_in, acc_in]
        out_specs += [
            pl.BlockSpec((1, 1), lambda i: (0, 0)),
            pl.BlockSpec((1, 1), lambda i: (0, 0)),
        ]
        out_shape += [
            jax.ShapeDtypeStruct((1, 1), jnp.float32),
            jax.ShapeDtypeStruct((1, 1), jnp.float32),
        ]
    return pl.pallas_call(
        functools.partial(_tc_body, nblk, total, first),
        grid=(nblk,),
        in_specs=in_specs,
        out_specs=out_specs,
        out_shape=out_shape,
        scratch_shapes=[
            pltpu.VMEM((ne, 1), jnp.float32),
            pltpu.VMEM((ne, 1), jnp.float32),
            pltpu.VMEM((ne, d), jnp.bfloat16),
            pltpu.VMEM((block_rows, 1), jnp.float32),
            pltpu.SMEM((2,), jnp.float32),
        ],
    )(*operands)


_CHUNK = 128


def _make_sc_gather(n, d):
    info = plsc.get_sparse_core_info()
    nw = info.num_cores * info.num_subcores
    b_per_w = n // nw
    n_chunks = b_per_w // _CHUNK
    mesh = plsc.VectorSubcoreMesh(core_axis_name="c", subcore_axis_name="s")

    @functools.partial(
        pl.kernel,
        mesh=mesh,
        out_type=jax.ShapeDtypeStruct((n, d), jnp.float32),
        scratch_types=[
            pltpu.VMEM((b_per_w,), jnp.int32),
            pltpu.VMEM((_CHUNK, d), jnp.float32),
            pltpu.VMEM((_CHUNK, d), jnp.float32),
            pltpu.SemaphoreType.DMA,
            pltpu.SemaphoreType.DMA,
            pltpu.SemaphoreType.DMA,
            pltpu.SemaphoreType.DMA,
        ],
    )
    def sc_gather(emb_hbm, ind_hbm, out_hbm, idx_v, q0, q1,
                  sg0, sg1, sw0, sw1):
        wid = lax.axis_index("s") * info.num_cores + lax.axis_index("c")
        base = wid * b_per_w
        pltpu.sync_copy(ind_hbm.at[pl.ds(base, b_per_w)], idx_v)
        q = [q0, q1]
        sg = [sg0, sg1]
        sw = [sw0, sw1]
        gather = [None, None]
        wb = [None, None]
        gather[0] = pltpu.async_copy(
            emb_hbm.at[idx_v.at[pl.ds(0, _CHUNK)]], q[0], sg[0])
        for ch in range(n_chunks):
            cur = ch & 1
            nxt = 1 - cur
            if ch + 1 < n_chunks:
                if wb[nxt] is not None:
                    wb[nxt].wait()
                gather[nxt] = pltpu.async_copy(
                    emb_hbm.at[idx_v.at[pl.ds((ch + 1) * _CHUNK, _CHUNK)]],
                    q[nxt], sg[nxt])
            gather[cur].wait()
            wb[cur] = pltpu.async_copy(
                q[cur], out_hbm.at[pl.ds(base + ch * _CHUNK, _CHUNK)],
                sw[cur])
        for h in wb:
            if h is not None:
                h.wait()

    return sc_gather


def kernel(input, input_mask, embed):
    t, b, d = input.shape
    n = t * b
    block_rows = 1024
    half = n // 2
    hblk = half // block_rows

    flatten = input.reshape(n, d)
    mask_flat = input_mask.reshape(n).astype(jnp.float32)
    xsq = jnp.sum(flatten * flatten, axis=1)

    f1, f2 = flatten[:half], flatten[half:]
    m1 = mask_flat[:half].reshape(hblk, 1, block_rows)
    m2 = mask_flat[half:].reshape(hblk, 1, block_rows)
    q1s = xsq[:half].reshape(hblk, 1, block_rows)
    q2s = xsq[half:].reshape(hblk, 1, block_rows)

    ind1, indg1, table, counts1, acc1 = _tc_stage(
        f1, embed, m1, q1s, block_rows, first=True)
    sc = _make_sc_gather(half, d)
    out1 = sc(table, indg1.reshape(half))

    ind2, indg2, diff, eff = _tc_stage(
        f2, embed, m2, q2s, block_rows, first=False,
        counts_in=counts1, acc_in=acc1)
    out2 = sc(table, indg2.reshape(half))

    q_st = jnp.concatenate([out1, out2], axis=0).reshape(t, b, d)
    ind = jnp.concatenate([ind1.reshape(half), ind2.reshape(half)])
    return (q_st, diff[0, 0], ind, eff[0, 0])


# speedup vs baseline: 1.4482x; 1.4482x over previous
import functools

import jax
import jax.numpy as jnp
from jax import lax
from jax.experimental import pallas as pl
from jax.experimental.pallas import tpu as pltpu
from jax.experimental.pallas import tpu_sc as plsc

_PAD_ROWS = 8


def _tc_body(nblk, total_elems, x_ref, e_ref, m_ref, xsq_ref, ind_ref,
             indg_ref, table_ref, diff_ref, eff_ref, counts_ref, esqc_ref,
             ebft_ref, ones_ref, acc_ref):
    i = pl.program_id(0)
    ne = e_ref.shape[1]
    r_ = x_ref.shape[0]

    @pl.when(i == 0)
    def _init():
        counts_ref[...] = jnp.zeros_like(counts_ref)
        acc_ref[0] = 0.0
        acc_ref[1] = 0.0
        e0 = e_ref[...]
        et = e0.T
        table_ref[0:ne, :] = et
        table_ref[ne:, :] = jnp.zeros_like(table_ref[ne:, :])
        ebft_ref[...] = (et + et).astype(jnp.bfloat16)
        esq = jnp.sum(e0 * e0, axis=0, keepdims=True)
        esqc_ref[...] = esq.reshape(ne, 1)
        ones_ref[...] = jnp.ones_like(ones_ref)

    x = x_ref[...]
    st2 = jnp.dot(ebft_ref[...], x.astype(jnp.bfloat16).T,
                  preferred_element_type=jnp.float32)
    xsq = xsq_ref[0]
    s = (st2 - xsq) - esqc_ref[...]
    smax = jnp.max(s, axis=0, keepdims=True)
    eq = s == smax
    iota = lax.broadcasted_iota(jnp.int32, (ne, r_), 0)
    rev = jnp.where(eq, jnp.int32(ne) - iota, 0)
    ind = jnp.int32(ne) - jnp.max(rev, axis=0)
    ind_ref[0, 0, :] = ind

    mrow = m_ref[0]
    indg_ref[0, 0, :] = jnp.where(mrow[0] > 0.0, ind, jnp.int32(ne))

    w = jnp.where(eq, mrow, 0.0)
    counts_ref[...] += jnp.dot(w, ones_ref[...],
                               preferred_element_type=jnp.float32)
    acc_ref[0] += jnp.sum(mrow[0] * (-smax[0]))
    acc_ref[1] += jnp.sum(mrow[0])

    @pl.when(i == nblk - 1)
    def _fin():
        diff_ref[...] = jnp.full((1, 1), acc_ref[0] / total_elems,
                                 jnp.float32)
        mcount = jnp.maximum(acc_ref[1], 1.0)
        mu = counts_ref[...] / mcount
        eff_ref[...] = jnp.full((1, 1), 1.0, jnp.float32) / jnp.sum(mu * mu)


def _tc_stage(flatten, embed, mask3, xsq3, block_rows):
    n, d = flatten.shape
    ne = embed.shape[1]
    nblk = n // block_rows
    total = float(n * d)
    return pl.pallas_call(
        functools.partial(_tc_body, nblk, total),
        grid=(nblk,),
        in_specs=[
            pl.BlockSpec((block_rows, d), lambda i: (i, 0)),
            pl.BlockSpec((d, ne), lambda i: (0, 0)),
            pl.BlockSpec((1, 1, block_rows), lambda i: (i, 0, 0)),
            pl.BlockSpec((1, 1, block_rows), lambda i: (i, 0, 0)),
        ],
        out_specs=[
            pl.BlockSpec((1, 1, block_rows), lambda i: (i, 0, 0)),
            pl.BlockSpec((1, 1, block_rows), lambda i: (i, 0, 0)),
            pl.BlockSpec((ne + _PAD_ROWS, d), lambda i: (0, 0)),
            pl.BlockSpec((1, 1), lambda i: (0, 0)),
            pl.BlockSpec((1, 1), lambda i: (0, 0)),
        ],
        out_shape=[
            jax.ShapeDtypeStruct((nblk, 1, block_rows), jnp.int32),
            jax.ShapeDtypeStruct((nblk, 1, block_rows), jnp.int32),
            jax.ShapeDtypeStruct((ne + _PAD_ROWS, d), jnp.float32),
            jax.ShapeDtypeStruct((1, 1), jnp.float32),
            jax.ShapeDtypeStruct((1, 1), jnp.float32),
        ],
        scratch_shapes=[
            pltpu.VMEM((ne, 1), jnp.float32),
            pltpu.VMEM((ne, 1), jnp.float32),
            pltpu.VMEM((ne, d), jnp.bfloat16),
            pltpu.VMEM((block_rows, 1), jnp.float32),
            pltpu.SMEM((2,), jnp.float32),
        ],
    )(flatten, embed, mask3, xsq3)


_CHUNK = 128


def _make_sc_gather(n, d):
    info = plsc.get_sparse_core_info()
    nw = info.num_cores * info.num_subcores
    b_per_w = n // nw
    n_chunks = b_per_w // _CHUNK
    mesh = plsc.VectorSubcoreMesh(core_axis_name="c", subcore_axis_name="s")

    @functools.partial(
        pl.kernel,
        mesh=mesh,
        out_type=jax.ShapeDtypeStruct((n, d), jnp.float32),
        scratch_types=[
            pltpu.VMEM((b_per_w,), jnp.int32),
            pltpu.VMEM((_CHUNK, d), jnp.float32),
            pltpu.VMEM((_CHUNK, d), jnp.float32),
            pltpu.SemaphoreType.DMA,
            pltpu.SemaphoreType.DMA,
            pltpu.SemaphoreType.DMA,
            pltpu.SemaphoreType.DMA,
        ],
    )
    def sc_gather(emb_hbm, ind_hbm, out_hbm, idx_v, q0, q1,
                  sg0, sg1, sw0, sw1):
        wid = lax.axis_index("s") * info.num_cores + lax.axis_index("c")
        base = wid * b_per_w
        pltpu.sync_copy(ind_hbm.at[pl.ds(base, b_per_w)], idx_v)
        q = [q0, q1]
        sg = [sg0, sg1]
        sw = [sw0, sw1]
        gather = [None, None]
        wb = [None, None]
        gather[0] = pltpu.async_copy(
            emb_hbm.at[idx_v.at[pl.ds(0, _CHUNK)]], q[0], sg[0])
        for ch in range(n_chunks):
            cur = ch & 1
            nxt = 1 - cur
            if ch + 1 < n_chunks:
                if wb[nxt] is not None:
                    wb[nxt].wait()
                gather[nxt] = pltpu.async_copy(
                    emb_hbm.at[idx_v.at[pl.ds((ch + 1) * _CHUNK, _CHUNK)]],
                    q[nxt], sg[nxt])
            gather[cur].wait()
            wb[cur] = pltpu.async_copy(
                q[cur], out_hbm.at[pl.ds(base + ch * _CHUNK, _CHUNK)],
                sw[cur])
        wb[0].wait()
        wb[1].wait()

    return sc_gather


def kernel(input, input_mask, embed):
    t, b, d = input.shape
    n = t * b
    block_rows = 1024

    flatten = input.reshape(n, d)
    mask_flat = input_mask.reshape(n).astype(jnp.float32)
    mask3 = mask_flat.reshape(n // block_rows, 1, block_rows)
    xsq3 = jnp.sum(flatten * flatten, axis=1).reshape(
        n // block_rows, 1, block_rows)

    ind3, indg3, table, diff, eff = _tc_stage(flatten, embed, mask3, xsq3,
                                              block_rows)
    ind = ind3.reshape(n)
    ind_gather = indg3.reshape(n)

    q_st = _make_sc_gather(n, d)(table, ind_gather)

    return (q_st.reshape(t, b, d), diff[0, 0], ind, eff[0, 0])


# speedup vs baseline: 1.4919x; 1.0302x over previous
import functools

import jax
import jax.numpy as jnp
from jax import lax
from jax.experimental import pallas as pl
from jax.experimental.pallas import tpu as pltpu
from jax.experimental.pallas import tpu_sc as plsc

_PAD_ROWS = 8


def _tc_body(nblk, total_elems, x_ref, e_ref, m_ref, xsq_ref, ind_ref,
             indg_ref, table_ref, diff_ref, eff_ref, counts_ref, esqc_ref,
             ebft_ref, ones_ref, acc_ref):
    i = pl.program_id(0)
    ne = e_ref.shape[1]
    r_ = x_ref.shape[0]

    @pl.when(i == 0)
    def _init():
        counts_ref[...] = jnp.zeros_like(counts_ref)
        acc_ref[0] = 0.0
        acc_ref[1] = 0.0
        e0 = e_ref[...]
        et = e0.T
        table_ref[0:ne, :] = et
        table_ref[ne:, :] = jnp.zeros_like(table_ref[ne:, :])
        ebft_ref[...] = (et + et).astype(jnp.bfloat16)
        esq = jnp.sum(e0 * e0, axis=0, keepdims=True)
        esqc_ref[...] = esq.reshape(ne, 1)
        ones_ref[...] = jnp.ones_like(ones_ref)

    x = x_ref[...]
    st2 = jnp.dot(ebft_ref[...], x.astype(jnp.bfloat16).T,
                  preferred_element_type=jnp.float32)
    xsq = xsq_ref[0]
    s = (st2 - xsq) - esqc_ref[...]
    smax = jnp.max(s, axis=0, keepdims=True)
    eq = s == smax
    iota = lax.broadcasted_iota(jnp.int32, (ne, r_), 0)
    rev = jnp.where(eq, jnp.int32(ne) - iota, 0)
    ind = jnp.int32(ne) - jnp.max(rev, axis=0)
    ind_ref[0, 0, :] = ind

    mrow = m_ref[0]
    indg_ref[0, 0, :] = jnp.where(mrow[0] > 0.0, ind, jnp.int32(ne))

    w = jnp.where(eq, mrow, 0.0)
    counts_ref[...] += jnp.dot(w, ones_ref[...],
                               preferred_element_type=jnp.float32)
    acc_ref[0] += jnp.sum(mrow[0] * (-smax[0]))
    acc_ref[1] += jnp.sum(mrow[0])

    @pl.when(i == nblk - 1)
    def _fin():
        diff_ref[...] = jnp.full((1, 1), acc_ref[0] / total_elems,
                                 jnp.float32)
        mcount = jnp.maximum(acc_ref[1], 1.0)
        mu = counts_ref[...] / mcount
        eff_ref[...] = jnp.full((1, 1), 1.0, jnp.float32) / jnp.sum(mu * mu)


def _tc_stage(flatten, embed, mask3, xsq3, block_rows):
    n, d = flatten.shape
    ne = embed.shape[1]
    nblk = n // block_rows
    total = float(n * d)
    return pl.pallas_call(
        functools.partial(_tc_body, nblk, total),
        grid=(nblk,),
        in_specs=[
            pl.BlockSpec((block_rows, d), lambda i: (i, 0)),
            pl.BlockSpec((d, ne), lambda i: (0, 0)),
            pl.BlockSpec((1, 1, block_rows), lambda i: (i, 0, 0)),
            pl.BlockSpec((1, 1, block_rows), lambda i: (i, 0, 0)),
        ],
        out_specs=[
            pl.BlockSpec((1, 1, block_rows), lambda i: (i, 0, 0)),
            pl.BlockSpec((1, 1, block_rows), lambda i: (i, 0, 0)),
            pl.BlockSpec((ne + _PAD_ROWS, d), lambda i: (0, 0)),
            pl.BlockSpec((1, 1), lambda i: (0, 0)),
            pl.BlockSpec((1, 1), lambda i: (0, 0)),
        ],
        out_shape=[
            jax.ShapeDtypeStruct((nblk, 1, block_rows), jnp.int32),
            jax.ShapeDtypeStruct((nblk, 1, block_rows), jnp.int32),
            jax.ShapeDtypeStruct((ne + _PAD_ROWS, d), jnp.float32),
            jax.ShapeDtypeStruct((1, 1), jnp.float32),
            jax.ShapeDtypeStruct((1, 1), jnp.float32),
        ],
        scratch_shapes=[
            pltpu.VMEM((ne, 1), jnp.float32),
            pltpu.VMEM((ne, 1), jnp.float32),
            pltpu.VMEM((ne, d), jnp.bfloat16),
            pltpu.VMEM((block_rows, 1), jnp.float32),
            pltpu.SMEM((2,), jnp.float32),
        ],
    )(flatten, embed, mask3, xsq3)


_CHUNK = 128


def _make_sc_gather(n, d):
    info = plsc.get_sparse_core_info()
    nw = info.num_cores * info.num_subcores
    b_per_w = n // nw
    n_chunks = b_per_w // _CHUNK
    mesh = plsc.VectorSubcoreMesh(core_axis_name="c", subcore_axis_name="s")

    @functools.partial(
        pl.kernel,
        mesh=mesh,
        out_type=jax.ShapeDtypeStruct((n, d), jnp.float32),
        scratch_types=[
            pltpu.VMEM((b_per_w,), jnp.int32),
            pltpu.VMEM((_CHUNK, d), jnp.float32),
            pltpu.VMEM((_CHUNK, d), jnp.float32),
            pltpu.SemaphoreType.DMA,
            pltpu.SemaphoreType.DMA,
            pltpu.SemaphoreType.DMA,
            pltpu.SemaphoreType.DMA,
        ],
    )
    def sc_gather(emb_hbm, ind_hbm, out_hbm, idx_v, q0, q1,
                  sg0, sg1, sw0, sw1):
        wid = lax.axis_index("s") * info.num_cores + lax.axis_index("c")
        base = wid * b_per_w
        pltpu.sync_copy(ind_hbm.at[pl.ds(base, b_per_w)], idx_v)
        q = [q0, q1]
        sg = [sg0, sg1]
        sw = [sw0, sw1]
        gather = [None, None]
        wb = [None, None]
        gather[0] = pltpu.async_copy(
            emb_hbm.at[idx_v.at[pl.ds(0, _CHUNK)]], q[0], sg[0])
        for ch in range(n_chunks):
            cur = ch & 1
            nxt = 1 - cur
            if ch + 1 < n_chunks:
                if wb[nxt] is not None:
                    wb[nxt].wait()
                gather[nxt] = pltpu.async_copy(
                    emb_hbm.at[idx_v.at[pl.ds((ch + 1) * _CHUNK, _CHUNK)]],
                    q[nxt], sg[nxt])
            gather[cur].wait()
            wb[cur] = pltpu.async_copy(
                q[cur], out_hbm.at[pl.ds(base + ch * _CHUNK, _CHUNK)],
                sw[cur])
        wb[0].wait()
        wb[1].wait()

    return sc_gather


def kernel(input, input_mask, embed):
    t, b, d = input.shape
    n = t * b
    block_rows = 2048

    flatten = input.reshape(n, d)
    mask_flat = input_mask.reshape(n).astype(jnp.float32)
    mask3 = mask_flat.reshape(n // block_rows, 1, block_rows)
    xsq3 = jnp.sum(flatten * flatten, axis=1).reshape(
        n // block_rows, 1, block_rows)

    ind3, indg3, table, diff, eff = _tc_stage(flatten, embed, mask3, xsq3,
                                              block_rows)
    ind = ind3.reshape(n)
    ind_gather = indg3.reshape(n)

    q_st = _make_sc_gather(n, d)(table, ind_gather)

    return (q_st.reshape(t, b, d), diff[0, 0], ind, eff[0, 0])
